# baseline (speedup 1.0000x reference)
import functools

import jax
import jax.numpy as jnp
from jax import lax
from jax.experimental import pallas as pl
from jax.experimental.pallas import tpu as pltpu
from jax.experimental.pallas import tpu_sc as plsc

_P = 4
_BI = 128
_EI = 8


def _build_sc_kernel(B, L, ES, NW, NV):
    TB = B // _BI
    TE = ES // _EI
    units = L * TB
    per_w = units // NW
    P = _P
    assert per_w % P == 0 and per_w >= 2 * P
    G = (per_w - P) // P
    mesh = plsc.VectorSubcoreMesh(core_axis_name="c", subcore_axis_name="s")
    scratch = (
        [pltpu.VMEM((_BI, ES), jnp.float32)] * P
        + [pltpu.VMEM((ES * _BI,), jnp.float32)] * 2
        + [pltpu.VMEM((ES, NV), jnp.float32)]
        + [pltpu.VMEM((_BI,), jnp.int32)] * (3 * P)
        + [pltpu.SemaphoreType.DMA] * (3 * P)
        + [pltpu.SemaphoreType.DMA] * 2
        + [pltpu.SemaphoreType.DMA]
    )

    @functools.partial(
        pl.kernel,
        out_type=jax.ShapeDtypeStruct((L, TE, TB, _EI * _BI), jnp.float32),
        mesh=mesh,
        scratch_types=scratch,
        compiler_params=pltpu.CompilerParams(use_tc_tiling_on_sc=False,
                                             needs_layout_passes=False),
    )
    def k(nt4, gt4, li4, nteT, gte, lne, out, *scr):
        rows = scr[0:P]
        tbufs = scr[P: P + 2]
        ntbl = scr[P + 2]
        idxs = [scr[P + 3 + 3 * b: P + 3 + 3 * b + 3] for b in range(P)]
        si = scr[4 * P + 3: 5 * P + 3]
        sga = scr[5 * P + 3: 6 * P + 3]
        sbc = scr[6 * P + 3: 7 * P + 3]
        ss = scr[7 * P + 3: 7 * P + 5]
        snt = scr[7 * P + 5]
        wid = lax.axis_index("s") * 2 + lax.axis_index("c")
        u0 = wid * per_w
        iota16 = lax.iota(jnp.int32, 16)
        jvecs = [iota16 + j * 16 for j in range(_BI // 16)]
        cd = [(d + iota16) & 15 for d in range(16)]
        sd = [((d + iota16) & 15) * _BI + iota16 for d in range(16)]

        pltpu.async_copy(nteT, ntbl, snt)

        def unit_lb(u):
            l = u >> 5
            return l, u & (TB - 1)

        def scat_issue(t, u):
            l, tb = unit_lb(u)
            for te in range(TE):
                pltpu.async_copy(
                    tbufs[t].at[pl.ds(te * _EI * _BI, _EI * _BI)],
                    out.at[l, te, tb], ss[t])

        def scat_wait(t, u):
            l, tb = unit_lb(u)
            for te in range(TE):
                pltpu.make_async_copy(
                    tbufs[t].at[pl.ds(te * _EI * _BI, _EI * _BI)],
                    out.at[l, te, tb], ss[t]).wait()

        def idx_slices(u):
            l, tb = unit_lb(u)
            tl = l >> 3
            li = l & 7
            return (nt4.at[tl, tb, li], gt4.at[tl, tb, li],
                    li4.at[tl, tb, li])

        def stage_a(b, u):
            s0, s1, s2 = idx_slices(u)
            pltpu.async_copy(s0, idxs[b][0], si[b])
            pltpu.async_copy(s1, idxs[b][1], si[b])
            pltpu.async_copy(s2, idxs[b][2], si[b])

        def stage_g(b, u):
            s0, s1, s2 = idx_slices(u)
            pltpu.make_async_copy(s0, idxs[b][0], si[b]).wait()
            pltpu.make_async_copy(s1, idxs[b][1], si[b]).wait()
            pltpu.make_async_copy(s2, idxs[b][2], si[b]).wait()
            pltpu.async_copy(lne.at[idxs[b][2]], rows[b], sga[b])

        def stage_b(b):
            pltpu.make_async_copy(lne.at[idxs[b][2]], rows[b], sga[b]).wait()
            pltpu.async_copy(gte.at[idxs[b][1]], rows[b], sbc[b], add=True)

        def stage_t(b, t, u):
            pltpu.make_async_copy(gte.at[idxs[b][1]], rows[b], sbc[b]).wait()
            scat_wait(t, u)

            def e0_body(t4, carry):
                e0 = t4 * 16
                cvecs = [cd[d] + e0 for d in range(16)]
                for j in range(_BI // 16):
                    soff = e0 * _BI + j * 16
                    ntv = idxs[b][0][pl.ds(j * 16, 16)]
                    vs = [plsc.load_gather(rows[b], [jvecs[j], cvecs[d]])
                          + plsc.load_gather(ntbl, [cvecs[d], ntv])
                          for d in range(16)]
                    for d in range(16):
                        plsc.store_scatter(tbufs[t], [sd[d] + soff], vs[d])
                return carry

            lax.fori_loop(0, ES // 16, e0_body, None)
            scat_issue(t, u)

        pltpu.make_async_copy(nteT, ntbl, snt).wait()
        scat_issue(0, u0 + 0)
        scat_issue(1, u0 + 1)

        stage_a(0, u0 + 0)
        stage_a(1, u0 + 1)
        stage_g(0, u0 + 0)
        stage_a(2, u0 + 2)
        stage_g(1, u0 + 1)
        stage_b(0)
        stage_a(3, u0 + 3)
        stage_g(2, u0 + 2)
        stage_b(1)
        stage_t(0, 0, u0 + 0)

        def group(g, carry):
            ub = u0 + P + P * g
            for b in range(P):
                u = ub + b
                stage_a(b, u)
                stage_g((b + P - 1) % P, u - 1)
                stage_b((b + P - 2) % P)
                stage_t((b + P - 3) % P, (b + 1) & 1, u - 3)
            return carry

        lax.fori_loop(0, G, group, None)

        ul = u0 + per_w - 1
        stage_g((per_w - 1) % P, ul)
        stage_b((per_w - 2) % P)
        stage_t((per_w - 3) % P, (per_w - 3) & 1, ul - 2)
        stage_b((per_w - 1) % P)
        stage_t((per_w - 2) % P, (per_w - 2) & 1, ul - 1)
        stage_t((per_w - 1) % P, (per_w - 1) & 1, ul)
        scat_wait((per_w - 2) & 1, ul - 1)
        scat_wait((per_w - 1) & 1, ul)

    return k


def kernel(naming_types, group_types, line_ids, naming_type_embeddings,
           group_type_embeddings, lines_num_embeddings):
    B, L = naming_types.shape
    NV, ES = naming_type_embeddings.shape
    NW = 32

    def idx_view(a):
        return (a.T.reshape(L // 8, 8, B // 128, 128)
                .transpose(0, 2, 1, 3).astype(jnp.int32))

    out4 = _build_sc_kernel(B, L, ES, NW, NV)(
        idx_view(naming_types), idx_view(group_types), idx_view(line_ids),
        naming_type_embeddings.T, group_type_embeddings,
        lines_num_embeddings)
    out5 = out4.reshape(L, ES // _EI, B // _BI, _EI, _BI)
    return out5.transpose(2, 4, 0, 1, 3).reshape(B, L, ES)


# speedup vs baseline: 1.0568x; 1.0568x over previous
import functools

import jax
import jax.numpy as jnp
from jax import lax
from jax.experimental import pallas as pl
from jax.experimental.pallas import tpu as pltpu
from jax.experimental.pallas import tpu_sc as plsc

_P = 8
_BI = 128
_EI = 8


def _build_sc_kernel(B, L, ES, NW):
    TB = B // _BI
    TE = ES // _EI
    units = L * TB
    per_w = units // NW
    P = _P
    assert per_w % P == 0 and per_w >= 2 * P
    G = (per_w - P) // P
    mesh = plsc.VectorSubcoreMesh(core_axis_name="c", subcore_axis_name="s")
    scratch = (
        [pltpu.VMEM((_BI, ES), jnp.float32)] * P
        + [pltpu.VMEM((ES * _BI,), jnp.float32)] * 2
        + [pltpu.VMEM((_BI,), jnp.int32)] * (3 * P)
        + [pltpu.SemaphoreType.DMA] * (3 * P)
        + [pltpu.SemaphoreType.DMA] * 2
    )

    @functools.partial(
        pl.kernel,
        out_type=jax.ShapeDtypeStruct((L, TE, TB, _EI * _BI), jnp.float32),
        mesh=mesh,
        scratch_types=scratch,
        compiler_params=pltpu.CompilerParams(use_tc_tiling_on_sc=False,
                                             needs_layout_passes=False),
    )
    def k(nt4, gt4, li4, nte, gte, lne, out, *scr):
        rows = scr[0:P]
        tbufs = scr[P: P + 2]
        idxs = [scr[P + 2 + 3 * b: P + 2 + 3 * b + 3] for b in range(P)]
        si = scr[4 * P + 2: 5 * P + 2]
        sga = scr[5 * P + 2: 6 * P + 2]
        sbc = scr[6 * P + 2: 7 * P + 2]
        ss = scr[7 * P + 2: 7 * P + 4]
        wid = lax.axis_index("s") * 2 + lax.axis_index("c")
        u0 = wid * per_w
        iota16 = lax.iota(jnp.int32, 16)
        jvecs = [iota16 + j * 16 for j in range(_BI // 16)]
        cd = [(d + iota16) & 15 for d in range(16)]
        sd = [((d + iota16) & 15) * _BI + iota16 for d in range(16)]

        def unit_lb(u):
            l = u >> 5
            return l, u & (TB - 1)

        def scat_issue(t, u):
            l, tb = unit_lb(u)
            for te in range(TE):
                pltpu.async_copy(
                    tbufs[t].at[pl.ds(te * _EI * _BI, _EI * _BI)],
                    out.at[l, te, tb], ss[t])

        def scat_wait(t, u):
            l, tb = unit_lb(u)
            for te in range(TE):
                pltpu.make_async_copy(
                    tbufs[t].at[pl.ds(te * _EI * _BI, _EI * _BI)],
                    out.at[l, te, tb], ss[t]).wait()

        def idx_slices(u):
            l, tb = unit_lb(u)
            tl = l >> 3
            li = l & 7
            return (nt4.at[tl, tb, li], gt4.at[tl, tb, li],
                    li4.at[tl, tb, li])

        def stage_a(b, u):
            s0, s1, s2 = idx_slices(u)
            pltpu.async_copy(s0, idxs[b][0], si[b])
            pltpu.async_copy(s1, idxs[b][1], si[b])
            pltpu.async_copy(s2, idxs[b][2], si[b])

        def stage_g(b, u):
            s0, s1, s2 = idx_slices(u)
            pltpu.make_async_copy(s0, idxs[b][0], si[b]).wait()
            pltpu.make_async_copy(s1, idxs[b][1], si[b]).wait()
            pltpu.make_async_copy(s2, idxs[b][2], si[b]).wait()
            pltpu.async_copy(lne.at[idxs[b][2]], rows[b], sga[b])

        def stage_b(b):
            pltpu.make_async_copy(lne.at[idxs[b][2]], rows[b], sga[b]).wait()
            pltpu.async_copy(nte.at[idxs[b][0]], rows[b], sbc[b], add=True)
            pltpu.async_copy(gte.at[idxs[b][1]], rows[b], sbc[b], add=True)

        def stage_t(b, t, u):
            pltpu.make_async_copy(nte.at[idxs[b][0]], rows[b], sbc[b]).wait()
            pltpu.make_async_copy(gte.at[idxs[b][1]], rows[b], sbc[b]).wait()
            scat_wait(t, u)

            def e0_body(t8, carry):
                e0 = (t8 >> 1) * 16
                jb = (t8 & 1) * 64
                cvecs = [cd[d] + e0 for d in range(16)]
                for jq in range(_BI // 32):
                    jvec = iota16 + (jb + jq * 16)
                    soff = e0 * _BI + jb + jq * 16
                    vs = [plsc.load_gather(rows[b], [jvec, cvecs[d]])
                          for d in range(16)]
                    for d in range(16):
                        plsc.store_scatter(tbufs[t], [sd[d] + soff], vs[d])
                return carry

            lax.fori_loop(0, (ES // 16) * 2, e0_body, None)
            scat_issue(t, u)

        scat_issue(0, u0 + 0)
        scat_issue(1, u0 + 1)

        stage_a(0, u0 + 0)
        stage_a(1, u0 + 1)
        stage_g(0, u0 + 0)
        stage_a(2, u0 + 2)
        stage_g(1, u0 + 1)
        stage_b(0)
        stage_a(3, u0 + 3)
        stage_g(2, u0 + 2)
        stage_b(1)
        stage_t(0, 0, u0 + 0)
        stage_a(4, u0 + 4)
        stage_g(3, u0 + 3)
        stage_b(2)
        stage_t(1, 1, u0 + 1)
        stage_a(5, u0 + 5)
        stage_g(4, u0 + 4)
        stage_b(3)
        stage_t(2, 0, u0 + 2)
        stage_a(6, u0 + 6)
        stage_g(5, u0 + 5)
        stage_b(4)
        stage_t(3, 1, u0 + 3)
        stage_a(7, u0 + 7)
        stage_g(6, u0 + 6)
        stage_b(5)
        stage_t(4, 0, u0 + 4)

        def group(g, carry):
            ub = u0 + P + P * g
            for b in range(P):
                u = ub + b
                stage_a(b, u)
                stage_g((b + P - 1) % P, u - 1)
                stage_b((b + P - 2) % P)
                stage_t((b + P - 3) % P, (b + 1) & 1, u - 3)
            return carry

        lax.fori_loop(0, G, group, None)

        ul = u0 + per_w - 1
        stage_g((per_w - 1) % P, ul)
        stage_b((per_w - 2) % P)
        stage_t((per_w - 3) % P, (per_w - 3) & 1, ul - 2)
        stage_b((per_w - 1) % P)
        stage_t((per_w - 2) % P, (per_w - 2) & 1, ul - 1)
        stage_t((per_w - 1) % P, (per_w - 1) & 1, ul)
        scat_wait((per_w - 2) & 1, ul - 1)
        scat_wait((per_w - 1) & 1, ul)

    return k


def kernel(naming_types, group_types, line_ids, naming_type_embeddings,
           group_type_embeddings, lines_num_embeddings):
    B, L = naming_types.shape
    ES = naming_type_embeddings.shape[1]
    NW = 32

    def idx_view(a):
        return (a.T.reshape(L // 8, 8, B // 128, 128)
                .transpose(0, 2, 1, 3).astype(jnp.int32))

    out4 = _build_sc_kernel(B, L, ES, NW)(
        idx_view(naming_types), idx_view(group_types), idx_view(line_ids),
        naming_type_embeddings, group_type_embeddings, lines_num_embeddings)
    out5 = out4.reshape(L, ES // _EI, B // _BI, _EI, _BI)
    return out5.transpose(2, 4, 0, 1, 3).reshape(B, L, ES)
